# baseline (speedup 1.0000x reference)
import functools

import jax
import jax.numpy as jnp
from jax import lax
from jax.experimental import pallas as pl
from jax.experimental.pallas import tpu as pltpu
from jax.experimental.pallas import tpu_sc as plsc

N = 10000
E = 160000
D = 768
H = 256

NSUB = 16
CHUNK = 64
EPT = 10240
NCHUNK = EPT // CHUNK
E_PAD = EPT * NSUB
ROWS = 10240
RPT = ROWS // NSUB
NBUF = 4


def _stage_a_body(x_ref, wemb_ref, bemb_ref, wgat_ref, al_ref, ar_ref,
                  wres_ref, feat2_ref, el_ref, er_ref, res_ref):
    x = x_ref[...]
    h = jnp.dot(x, wemb_ref[...], preferred_element_type=jnp.float32)
    h = h + bemb_ref[...]
    h = jnp.where(h > 0, h, jnp.exp(h) - 1.0)
    feat = jnp.dot(h, wgat_ref[...], preferred_element_type=jnp.float32)
    res_ref[...] = jnp.dot(h, wres_ref[...], preferred_element_type=jnp.float32)
    feat2_ref[0] = feat[:, :128]
    feat2_ref[1] = feat[:, 128:]
    el_ref[...] = jnp.sum(feat * al_ref[...], axis=1, keepdims=True)
    er_ref[...] = jnp.sum(feat * ar_ref[...], axis=1, keepdims=True)


def _stage_a(features, W_emb, b_emb, W_gat, attn_l, attn_r, W_res):
    B = 1000
    grid = (N // B,)
    return pl.pallas_call(
        _stage_a_body,
        grid=grid,
        in_specs=[
            pl.BlockSpec((B, D), lambda i: (i, 0)),
            pl.BlockSpec((D, D), lambda i: (0, 0)),
            pl.BlockSpec((1, D), lambda i: (0, 0)),
            pl.BlockSpec((D, H), lambda i: (0, 0)),
            pl.BlockSpec((1, H), lambda i: (0, 0)),
            pl.BlockSpec((1, H), lambda i: (0, 0)),
            pl.BlockSpec((D, H), lambda i: (0, 0)),
        ],
        out_specs=[
            pl.BlockSpec((2, B, 128), lambda i: (0, i, 0)),
            pl.BlockSpec((B, 1), lambda i: (i, 0)),
            pl.BlockSpec((B, 1), lambda i: (i, 0)),
            pl.BlockSpec((B, H), lambda i: (i, 0)),
        ],
        out_shape=[
            jax.ShapeDtypeStruct((2, N, 128), jnp.float32),
            jax.ShapeDtypeStruct((N, 1), jnp.float32),
            jax.ShapeDtypeStruct((N, 1), jnp.float32),
            jax.ShapeDtypeStruct((N, H), jnp.float32),
        ],
    )(features, W_emb, b_emb.reshape(1, D), W_gat, attn_l.reshape(1, H),
      attn_r.reshape(1, H), W_res)


SUPER = 4
NSUPER = NCHUNK // SUPER


def _stage_b_body(feat2_h, el_h, er_h, src_h, dst_h, z2_h, z1_h,
                  acc0_o, acc1_o, den0_o, den1_o,
                  sidx_v, didx_v, cidx_v, elg_v, erg_v, rows_v, ex_v,
                  acc_sh, den_sh,
                  sem_g, sem_el, sem_er, sem_sc, sem_den, sem_si, sem_di):
    c = lax.axis_index("c")
    s = lax.axis_index("s")
    base = s * RPT
    coff = c * N
    pltpu.sync_copy(z2_h, acc_sh.at[pl.ds(base, RPT)])
    pltpu.sync_copy(z1_h, den_sh.at[pl.ds(base, RPT)])
    pltpu.sync_copy(src_h.at[s, pl.ds(0, SUPER)], sidx_v.at[0])
    pltpu.sync_copy(dst_h.at[s, pl.ds(0, SUPER)], didx_v.at[0])
    plsc.subcore_barrier()

    def issue_gathers(qq, j8, b):
        for g in range(CHUNK // 16):
            sl = pl.ds(g * 16, 16)
            cidx_v[b, sl] = sidx_v[qq, j8, sl] + coff
        pltpu.async_copy(feat2_h.at[cidx_v.at[b]], rows_v.at[b], sem_g.at[b])
        pltpu.async_copy(el_h.at[sidx_v.at[qq, j8]], elg_v.at[b],
                         sem_el.at[b])
        pltpu.async_copy(er_h.at[didx_v.at[qq, j8]], erg_v.at[b],
                         sem_er.at[b])

    def wait_gathers(b):
        pltpu.make_async_copy(feat2_h.at[cidx_v.at[b]], rows_v.at[b],
                              sem_g.at[b]).wait()
        pltpu.make_async_copy(el_h.at[cidx_v.at[b]], elg_v.at[b],
                              sem_el.at[b]).wait()
        pltpu.make_async_copy(er_h.at[cidx_v.at[b]], erg_v.at[b],
                              sem_er.at[b]).wait()

    def wait_scatters(b, didx_row, den_core):
        pltpu.make_async_copy(rows_v.at[b], acc_sh.at[didx_row],
                              sem_sc.at[b]).wait()

        @pl.when(c == den_core)
        def _():
            pltpu.make_async_copy(ex_v.at[b], den_sh.at[didx_row],
                                  sem_den.at[b]).wait()

    issue_gathers(0, 0, 0)
    issue_gathers(0, 1, 1)

    def outer(j2, carry):
        q = j2 & 1
        for j8 in range(SUPER):
            b = j8
            bp2 = (j8 + 2) % NBUF
            if j8 <= 1:
                @pl.when(j2 > 0)
                def _():
                    wait_scatters(bp2, didx_v.at[1 - q, j8 + 2],
                                  (j8 + 2) & 1)
            else:
                wait_scatters(bp2, didx_v.at[q, j8 - 2], (j8 - 2) & 1)
            if j8 == 0:
                @pl.when(j2 < NSUPER - 1)
                def _():
                    pltpu.async_copy(
                        src_h.at[s, pl.ds((j2 + 1) * SUPER, SUPER)],
                        sidx_v.at[1 - q], sem_si.at[1 - q])
                    pltpu.async_copy(
                        dst_h.at[s, pl.ds((j2 + 1) * SUPER, SUPER)],
                        didx_v.at[1 - q], sem_di.at[1 - q])
            if j8 <= 1:
                issue_gathers(q, j8 + 2, bp2)
            elif j8 == 2:
                @pl.when(j2 < NSUPER - 1)
                def _():
                    pltpu.make_async_copy(
                        src_h.at[s, pl.ds((j2 + 1) * SUPER, SUPER)],
                        sidx_v.at[1 - q], sem_si.at[1 - q]).wait()
                    pltpu.make_async_copy(
                        dst_h.at[s, pl.ds((j2 + 1) * SUPER, SUPER)],
                        didx_v.at[1 - q], sem_di.at[1 - q]).wait()
                    issue_gathers(1 - q, 0, bp2)
            else:
                @pl.when(j2 < NSUPER - 1)
                def _():
                    issue_gathers(1 - q, 1, bp2)
            wait_gathers(b)
            for g in range(CHUNK // 16):
                sl = pl.ds(g * 16, 16)
                z = elg_v[b, sl] + erg_v[b, sl]
                z = jnp.maximum(z, 0.2 * z)
                ex_v[b, sl] = jnp.exp(z)

            def scale_body(gg, _):
                exg = ex_v[b, pl.ds(gg * 16, 16)]
                for l in range(16):
                    w = exg[l]
                    e = gg * 16 + l
                    for k in range(128 // 16):
                        ks = pl.ds(k * 16, 16)
                        rows_v[b, e, ks] = rows_v[b, e, ks] * w
                return 0

            lax.fori_loop(0, CHUNK // 16, scale_body, 0)
            pltpu.async_copy(rows_v.at[b], acc_sh.at[didx_v.at[q, j8]],
                             sem_sc.at[b], add=True)

            @pl.when(c == (j8 & 1))
            def _():
                pltpu.async_copy(ex_v.at[b], den_sh.at[didx_v.at[q, j8]],
                                 sem_den.at[b], add=True)
        return 0

    lax.fori_loop(0, NSUPER, outer, 0)
    qlast = (NSUPER - 1) & 1
    wait_scatters(2, didx_v.at[qlast, 2], 0)
    wait_scatters(3, didx_v.at[qlast, 3], 1)
    plsc.subcore_barrier()

    @pl.when(c == 0)
    def _():
        pltpu.sync_copy(acc_sh.at[pl.ds(base, RPT)], acc0_o.at[pl.ds(base, RPT)])
        pltpu.sync_copy(den_sh.at[pl.ds(base, RPT)], den0_o.at[pl.ds(base, RPT)])

    @pl.when(c == 1)
    def _():
        pltpu.sync_copy(acc_sh.at[pl.ds(base, RPT)], acc1_o.at[pl.ds(base, RPT)])
        pltpu.sync_copy(den_sh.at[pl.ds(base, RPT)], den1_o.at[pl.ds(base, RPT)])


def _stage_b(feat2, el1d, er1d, src_r, dst_r):
    mesh = plsc.VectorSubcoreMesh(core_axis_name="c", subcore_axis_name="s")
    zeros2d = jnp.zeros((RPT, 128), jnp.float32)
    zeros1d = jnp.zeros((RPT,), jnp.float32)
    k = pl.kernel(
        _stage_b_body,
        out_type=[
            jax.ShapeDtypeStruct((ROWS, 128), jnp.float32),
            jax.ShapeDtypeStruct((ROWS, 128), jnp.float32),
            jax.ShapeDtypeStruct((ROWS,), jnp.float32),
            jax.ShapeDtypeStruct((ROWS,), jnp.float32),
        ],
        mesh=mesh,
        compiler_params=pltpu.CompilerParams(needs_layout_passes=False),
        scratch_types=[
            pltpu.VMEM((2, SUPER, CHUNK), jnp.int32),
            pltpu.VMEM((2, SUPER, CHUNK), jnp.int32),
            pltpu.VMEM((NBUF, CHUNK), jnp.int32),
            pltpu.VMEM((NBUF, CHUNK), jnp.float32),
            pltpu.VMEM((NBUF, CHUNK), jnp.float32),
            pltpu.VMEM((NBUF, CHUNK, 128), jnp.float32),
            pltpu.VMEM((NBUF, CHUNK), jnp.float32),
            pltpu.VMEM_SHARED((ROWS, 128), jnp.float32),
            pltpu.VMEM_SHARED((ROWS,), jnp.float32),
            pltpu.SemaphoreType.DMA((NBUF,)),
            pltpu.SemaphoreType.DMA((NBUF,)),
            pltpu.SemaphoreType.DMA((NBUF,)),
            pltpu.SemaphoreType.DMA((NBUF,)),
            pltpu.SemaphoreType.DMA((NBUF,)),
            pltpu.SemaphoreType.DMA((2,)),
            pltpu.SemaphoreType.DMA((2,)),
        ],
    )
    return k(feat2, el1d, er1d, src_r, dst_r, zeros2d, zeros1d)


def _stage_c_body(acc0_ref, acc1_ref, den0_ref, den1_ref, res_ref, bgat_ref,
                  w1_ref, b1_ref, w2_ref, b2_ref, out_ref, psum_ref):
    i = pl.program_id(0)

    @pl.when(i == 0)
    def _():
        psum_ref[...] = jnp.zeros_like(psum_ref)

    acc = jnp.concatenate([acc0_ref[...], acc1_ref[...]], axis=1)
    den = den0_ref[...] + den1_ref[...] + 1e-9
    o = acc / den + res_ref[...] + bgat_ref[...]
    o = jnp.where(o > 0, o, jnp.exp(o) - 1.0)
    psum_ref[...] += jnp.sum(o, axis=0, keepdims=True)

    @pl.when(i == pl.num_programs(0) - 1)
    def _():
        pooled = psum_ref[...] / N
        z = jnp.dot(pooled, w1_ref[...], preferred_element_type=jnp.float32)
        z = jnp.maximum(z + b1_ref[...], 0.0)
        out_ref[...] = jnp.dot(z, w2_ref[...],
                               preferred_element_type=jnp.float32) + b2_ref[...]


def _stage_c(acc0, acc1, den0, den1, res, b_gat, W1, b1, W2, b2):
    B = 1000
    grid = (N // B,)
    return pl.pallas_call(
        _stage_c_body,
        grid=grid,
        in_specs=[
            pl.BlockSpec((B, 128), lambda i: (i, 0)),
            pl.BlockSpec((B, 128), lambda i: (i, 0)),
            pl.BlockSpec((B, 1), lambda i: (i, 0)),
            pl.BlockSpec((B, 1), lambda i: (i, 0)),
            pl.BlockSpec((B, H), lambda i: (i, 0)),
            pl.BlockSpec((1, H), lambda i: (0, 0)),
            pl.BlockSpec((H, 128), lambda i: (0, 0)),
            pl.BlockSpec((1, 128), lambda i: (0, 0)),
            pl.BlockSpec((128, 2), lambda i: (0, 0)),
            pl.BlockSpec((1, 2), lambda i: (0, 0)),
        ],
        out_specs=pl.BlockSpec((1, 2), lambda i: (0, 0)),
        out_shape=jax.ShapeDtypeStruct((1, 2), jnp.float32),
        scratch_shapes=[pltpu.VMEM((1, H), jnp.float32)],
    )(acc0, acc1, den0, den1, res, b_gat.reshape(1, H), W1,
      b1.reshape(1, 128), W2, b2.reshape(1, 2))


def kernel(features, edge_index, W_emb, b_emb, W_gat, attn_l, attn_r, W_res,
           b_gat, W1, b1, W2, b2):
    feat2, el, er, res = _stage_a(
        features, W_emb, b_emb, W_gat, attn_l, attn_r, W_res)

    src = edge_index[0]
    dst = edge_index[1]
    pad = E_PAD - E
    src_p = jnp.concatenate([src, jnp.zeros((pad,), jnp.int32)])
    dst_p = jnp.concatenate([dst, jnp.full((pad,), N, jnp.int32)])
    src_r = src_p.reshape(NSUB, NCHUNK, CHUNK)
    dst_r = dst_p.reshape(NSUB, NCHUNK, CHUNK)

    er_pad = jnp.concatenate([er[:, 0], jnp.zeros((ROWS - N,), jnp.float32)])
    acc0, acc1, den0, den1 = _stage_b(feat2.reshape(2 * N, 128), el[:, 0],
                                      er_pad, src_r, dst_r)

    return _stage_c(acc0, acc1, den0.reshape(ROWS, 1), den1.reshape(ROWS, 1),
                    res, b_gat, W1, b1, W2, b2)


# speedup vs baseline: 1.0274x; 1.0274x over previous
import functools

import jax
import jax.numpy as jnp
from jax import lax
from jax.experimental import pallas as pl
from jax.experimental.pallas import tpu as pltpu
from jax.experimental.pallas import tpu_sc as plsc

N = 10000
E = 160000
D = 768
H = 256

NSUB = 16
CHUNK = 128
EPT = 10240
NCHUNK = EPT // CHUNK
E_PAD = EPT * NSUB
ROWS = 10240
RPT = ROWS // NSUB


def _stage_a_body(x_ref, wemb_ref, bemb_ref, wgat_ref, al_ref, ar_ref,
                  wres_ref, feat2_ref, el_ref, er_ref, res_ref):
    x = x_ref[...]
    h = jnp.dot(x, wemb_ref[...], preferred_element_type=jnp.float32)
    h = h + bemb_ref[...]
    h = jnp.where(h > 0, h, jnp.exp(h) - 1.0)
    feat = jnp.dot(h, wgat_ref[...], preferred_element_type=jnp.float32)
    res_ref[...] = jnp.dot(h, wres_ref[...], preferred_element_type=jnp.float32)
    feat2_ref[0] = feat[:, :128]
    feat2_ref[1] = feat[:, 128:]
    el_ref[...] = jnp.sum(feat * al_ref[...], axis=1, keepdims=True)
    er_ref[...] = jnp.sum(feat * ar_ref[...], axis=1, keepdims=True)


def _stage_a(features, W_emb, b_emb, W_gat, attn_l, attn_r, W_res):
    B = 1000
    grid = (N // B,)
    return pl.pallas_call(
        _stage_a_body,
        grid=grid,
        in_specs=[
            pl.BlockSpec((B, D), lambda i: (i, 0)),
            pl.BlockSpec((D, D), lambda i: (0, 0)),
            pl.BlockSpec((1, D), lambda i: (0, 0)),
            pl.BlockSpec((D, H), lambda i: (0, 0)),
            pl.BlockSpec((1, H), lambda i: (0, 0)),
            pl.BlockSpec((1, H), lambda i: (0, 0)),
            pl.BlockSpec((D, H), lambda i: (0, 0)),
        ],
        out_specs=[
            pl.BlockSpec((2, B, 128), lambda i: (0, i, 0)),
            pl.BlockSpec((B, 1), lambda i: (i, 0)),
            pl.BlockSpec((B, 1), lambda i: (i, 0)),
            pl.BlockSpec((B, H), lambda i: (i, 0)),
        ],
        out_shape=[
            jax.ShapeDtypeStruct((2, N, 128), jnp.float32),
            jax.ShapeDtypeStruct((N, 1), jnp.float32),
            jax.ShapeDtypeStruct((N, 1), jnp.float32),
            jax.ShapeDtypeStruct((N, H), jnp.float32),
        ],
    )(features, W_emb, b_emb.reshape(1, D), W_gat, attn_l.reshape(1, H),
      attn_r.reshape(1, H), W_res)


SUPER = 8
NSUPER = NCHUNK // SUPER


def _stage_b_body(feat2_h, el_h, er_h, src_h, dst_h, z2_h, z1_h,
                  acc0_o, acc1_o, den_o,
                  sidx_v, didx_v, cidx_v, elg_v, erg_v, rows_v, ex_v,
                  acc_sh, den_sh,
                  sem_g, sem_el, sem_er, sem_sc, sem_den, sem_si, sem_di):
    c = lax.axis_index("c")
    s = lax.axis_index("s")
    base = s * RPT
    coff = c * N
    def issue_gathers(qq, j8, p):
        for g in range(CHUNK // 16):
            sl = pl.ds(g * 16, 16)
            cidx_v[p, sl] = sidx_v[qq, j8, sl] + coff
        pltpu.async_copy(feat2_h.at[cidx_v.at[p]], rows_v.at[p], sem_g.at[p])
        pltpu.async_copy(el_h.at[sidx_v.at[qq, j8]], elg_v.at[p],
                         sem_el.at[p])
        pltpu.async_copy(er_h.at[didx_v.at[qq, j8]], erg_v.at[p],
                         sem_er.at[p])

    def wait_gathers(p):
        pltpu.make_async_copy(feat2_h.at[cidx_v.at[p]], rows_v.at[p],
                              sem_g.at[p]).wait()
        pltpu.make_async_copy(el_h.at[cidx_v.at[p]], elg_v.at[p],
                              sem_el.at[p]).wait()
        pltpu.make_async_copy(er_h.at[cidx_v.at[p]], erg_v.at[p],
                              sem_er.at[p]).wait()

    def wait_scatters(p, didx_row):
        pltpu.make_async_copy(rows_v.at[p], acc_sh.at[didx_row],
                              sem_sc.at[p]).wait()

        @pl.when(c == 0)
        def _():
            pltpu.make_async_copy(ex_v.at[p], den_sh.at[didx_row],
                                  sem_den.at[p]).wait()

    pltpu.sync_copy(src_h.at[s, pl.ds(0, SUPER)], sidx_v.at[0])
    pltpu.sync_copy(dst_h.at[s, pl.ds(0, SUPER)], didx_v.at[0])
    issue_gathers(0, 0, 0)
    pltpu.sync_copy(z2_h, acc_sh.at[pl.ds(base, RPT)])
    pltpu.sync_copy(z1_h, den_sh.at[pl.ds(base, RPT)])
    plsc.subcore_barrier()

    def outer(j2, carry):
        q = j2 & 1
        for j8 in range(SUPER):
            p = j8 & 1
            if j8 == 0:
                @pl.when(j2 > 0)
                def _():
                    wait_scatters(1, didx_v.at[1 - q, SUPER - 1])

                @pl.when(j2 < NSUPER - 1)
                def _():
                    pltpu.async_copy(
                        src_h.at[s, pl.ds((j2 + 1) * SUPER, SUPER)],
                        sidx_v.at[1 - q], sem_si.at[1 - q])
                    pltpu.async_copy(
                        dst_h.at[s, pl.ds((j2 + 1) * SUPER, SUPER)],
                        didx_v.at[1 - q], sem_di.at[1 - q])
            else:
                wait_scatters(1 - p, didx_v.at[q, j8 - 1])
            if j8 < SUPER - 1:
                issue_gathers(q, j8 + 1, 1 - p)
            else:
                @pl.when(j2 < NSUPER - 1)
                def _():
                    pltpu.make_async_copy(
                        src_h.at[s, pl.ds((j2 + 1) * SUPER, SUPER)],
                        sidx_v.at[1 - q], sem_si.at[1 - q]).wait()
                    pltpu.make_async_copy(
                        dst_h.at[s, pl.ds((j2 + 1) * SUPER, SUPER)],
                        didx_v.at[1 - q], sem_di.at[1 - q]).wait()
                    issue_gathers(1 - q, 0, 1 - p)
            wait_gathers(p)
            for g in range(CHUNK // 16):
                sl = pl.ds(g * 16, 16)
                z = elg_v[p, sl] + erg_v[p, sl]
                z = jnp.maximum(z, 0.2 * z)
                ex_v[p, sl] = jnp.exp(z)

            def scale_body(gg, _):
                exg = ex_v[p, pl.ds(gg * 16, 16)]
                for l in range(16):
                    w = exg[l]
                    e = gg * 16 + l
                    for k in range(128 // 16):
                        ks = pl.ds(k * 16, 16)
                        rows_v[p, e, ks] = rows_v[p, e, ks] * w
                return 0

            lax.fori_loop(0, CHUNK // 16, scale_body, 0)
            pltpu.async_copy(rows_v.at[p], acc_sh.at[didx_v.at[q, j8]],
                             sem_sc.at[p], add=True)

            @pl.when(c == 0)
            def _():
                pltpu.async_copy(ex_v.at[p], den_sh.at[didx_v.at[q, j8]],
                                 sem_den.at[p], add=True)
        return 0

    lax.fori_loop(0, NSUPER, outer, 0)
    wait_scatters(1, didx_v.at[(NSUPER - 1) & 1, SUPER - 1])
    plsc.subcore_barrier()

    @pl.when(c == 0)
    def _():
        pltpu.sync_copy(acc_sh.at[pl.ds(base, RPT)], acc0_o.at[pl.ds(base, RPT)])
        pltpu.sync_copy(den_sh.at[pl.ds(base, RPT)], den_o.at[pl.ds(base, RPT)])

    @pl.when(c == 1)
    def _():
        pltpu.sync_copy(acc_sh.at[pl.ds(base, RPT)], acc1_o.at[pl.ds(base, RPT)])


def _stage_b(feat2, el1d, er1d, src_r, dst_r):
    mesh = plsc.VectorSubcoreMesh(core_axis_name="c", subcore_axis_name="s")
    zeros2d = jnp.zeros((RPT, 128), jnp.float32)
    zeros1d = jnp.zeros((RPT,), jnp.float32)
    k = pl.kernel(
        _stage_b_body,
        out_type=[
            jax.ShapeDtypeStruct((ROWS, 128), jnp.float32),
            jax.ShapeDtypeStruct((ROWS, 128), jnp.float32),
            jax.ShapeDtypeStruct((ROWS,), jnp.float32),
        ],
        mesh=mesh,
        compiler_params=pltpu.CompilerParams(needs_layout_passes=False),
        scratch_types=[
            pltpu.VMEM((2, SUPER, CHUNK), jnp.int32),
            pltpu.VMEM((2, SUPER, CHUNK), jnp.int32),
            pltpu.VMEM((2, CHUNK), jnp.int32),
            pltpu.VMEM((2, CHUNK), jnp.float32),
            pltpu.VMEM((2, CHUNK), jnp.float32),
            pltpu.VMEM((2, CHUNK, 128), jnp.float32),
            pltpu.VMEM((2, CHUNK), jnp.float32),
            pltpu.VMEM_SHARED((ROWS, 128), jnp.float32),
            pltpu.VMEM_SHARED((ROWS,), jnp.float32),
            pltpu.SemaphoreType.DMA((2,)),
            pltpu.SemaphoreType.DMA((2,)),
            pltpu.SemaphoreType.DMA((2,)),
            pltpu.SemaphoreType.DMA((2,)),
            pltpu.SemaphoreType.DMA((2,)),
            pltpu.SemaphoreType.DMA((2,)),
            pltpu.SemaphoreType.DMA((2,)),
        ],
    )
    return k(feat2, el1d, er1d, src_r, dst_r, zeros2d, zeros1d)


def _stage_c_body(acc0_ref, acc1_ref, den_ref, res_ref, bgat_ref,
                  w1_ref, b1_ref, w2_ref, b2_ref, out_ref, psum_ref):
    i = pl.program_id(0)

    @pl.when(i == 0)
    def _():
        psum_ref[...] = jnp.zeros_like(psum_ref)

    acc = jnp.concatenate([acc0_ref[...], acc1_ref[...]], axis=1)
    den = den_ref[...] + 1e-9
    o = acc / den + res_ref[...] + bgat_ref[...]
    o = jnp.where(o > 0, o, jnp.exp(o) - 1.0)
    psum_ref[...] += jnp.sum(o, axis=0, keepdims=True)

    @pl.when(i == pl.num_programs(0) - 1)
    def _():
        pooled = psum_ref[...] / N
        z = jnp.dot(pooled, w1_ref[...], preferred_element_type=jnp.float32)
        z = jnp.maximum(z + b1_ref[...], 0.0)
        out_ref[...] = jnp.dot(z, w2_ref[...],
                               preferred_element_type=jnp.float32) + b2_ref[...]


def _stage_c(acc0, acc1, den, res, b_gat, W1, b1, W2, b2):
    B = 1000
    grid = (N // B,)
    return pl.pallas_call(
        _stage_c_body,
        grid=grid,
        in_specs=[
            pl.BlockSpec((B, 128), lambda i: (i, 0)),
            pl.BlockSpec((B, 128), lambda i: (i, 0)),
            pl.BlockSpec((B, 1), lambda i: (i, 0)),
            pl.BlockSpec((B, H), lambda i: (i, 0)),
            pl.BlockSpec((1, H), lambda i: (0, 0)),
            pl.BlockSpec((H, 128), lambda i: (0, 0)),
            pl.BlockSpec((1, 128), lambda i: (0, 0)),
            pl.BlockSpec((128, 2), lambda i: (0, 0)),
            pl.BlockSpec((1, 2), lambda i: (0, 0)),
        ],
        out_specs=pl.BlockSpec((1, 2), lambda i: (0, 0)),
        out_shape=jax.ShapeDtypeStruct((1, 2), jnp.float32),
        scratch_shapes=[pltpu.VMEM((1, H), jnp.float32)],
    )(acc0, acc1, den, res, b_gat.reshape(1, H), W1, b1.reshape(1, 128),
      W2, b2.reshape(1, 2))


def kernel(features, edge_index, W_emb, b_emb, W_gat, attn_l, attn_r, W_res,
           b_gat, W1, b1, W2, b2):
    feat2, el, er, res = _stage_a(
        features, W_emb, b_emb, W_gat, attn_l, attn_r, W_res)

    src = edge_index[0]
    dst = edge_index[1]
    pad = E_PAD - E
    src_p = jnp.concatenate([src, jnp.zeros((pad,), jnp.int32)])
    dst_p = jnp.concatenate([dst, jnp.full((pad,), N, jnp.int32)])
    src_r = src_p.reshape(NSUB, NCHUNK, CHUNK)
    dst_r = dst_p.reshape(NSUB, NCHUNK, CHUNK)

    er_pad = jnp.concatenate([er[:, 0], jnp.zeros((ROWS - N,), jnp.float32)])
    acc0, acc1, den = _stage_b(feat2.reshape(2 * N, 128), el[:, 0], er_pad,
                               src_r, dst_r)

    return _stage_c(acc0, acc1, den.reshape(ROWS, 1), res, b_gat, W1, b1,
                    W2, b2)
